# baseline (speedup 1.0000x reference)
import functools

import jax
import jax.numpy as jnp
from jax import lax
from jax.experimental import pallas as pl
from jax.experimental.pallas import tpu as pltpu
from jax.experimental.pallas import tpu_sc as plsc

F32 = jnp.float32

_NC = 2
_NS = 16
_NW = _NC * _NS

_CH = 80


def _prep_body(node_ref, w1sr_ref, g_ref, w1g_ref, b1_ref, t_ref, c4_ref):
    n = node_ref[...]
    sr = jnp.dot(n, w1sr_ref[...], preferred_element_type=F32)
    t_ref[...] = jnp.concatenate(
        [sr, jnp.zeros((n.shape[0], 64), dtype=F32)], axis=1)
    c = jnp.dot(g_ref[...], w1g_ref[...], preferred_element_type=F32) + b1_ref[...]
    c4_ref[...] = jnp.concatenate([c, c, c, c], axis=1)


def _mlp_body(e4_ref, g_ref, w1e_ref, c4_ref, w2_ref, b24_ref, out_ref):
    x = jnp.dot(e4_ref[...], w1e_ref[...], preferred_element_type=F32)
    g = jnp.reshape(g_ref[...], x.shape)
    x = x + g + c4_ref[...]
    h = jnp.maximum(x, 0.0)
    out_ref[...] = jnp.dot(h, w2_ref[...], preferred_element_type=F32) + b24_ref[...]


def _make_gather(num_edges, latent):
    per_w = num_edges // _NW
    nchunk = per_w // _CH
    mesh = plsc.VectorSubcoreMesh(core_axis_name="c", subcore_axis_name="s")

    @functools.partial(
        pl.kernel,
        mesh=mesh,
        out_type=jax.ShapeDtypeStruct((num_edges * latent,), F32),
        scratch_types=[
            pltpu.VMEM((per_w,), jnp.int32),
            pltpu.VMEM((per_w,), jnp.int32),
            pltpu.VMEM((_CH, 4 * latent), F32),
            pltpu.VMEM((_CH, 4 * latent), F32),
            pltpu.VMEM((_CH, 4 * latent), F32),
            pltpu.VMEM((_CH, 4 * latent), F32),
            pltpu.VMEM((_CH * latent,), F32),
            pltpu.VMEM((_CH * latent,), F32),
            pltpu.SemaphoreType.DMA,
            pltpu.SemaphoreType.DMA,
            pltpu.SemaphoreType.DMA,
            pltpu.SemaphoreType.DMA,
        ],
    )
    def gather_call(t_hbm, src_hbm, dst_hbm, out_g,
                    idx_s, idx_d, buf_s0, buf_r0, buf_s1, buf_r1,
                    pk0, pk1, sem0, sem1, sem_w0, sem_w1):
        wid = lax.axis_index("s") * _NC + lax.axis_index("c")
        ebase = wid * per_w
        pltpu.sync_copy(src_hbm.at[pl.ds(ebase, per_w)], idx_s)
        pltpu.sync_copy(dst_hbm.at[pl.ds(ebase, per_w)], idx_d)

        bufs = ((buf_s0, buf_r0, pk0, sem0, sem_w0),
                (buf_s1, buf_r1, pk1, sem1, sem_w1))
        pending = {}
        wpending = {}

        def fire(k):
            buf_s, buf_r, _, sem, _ = bufs[k % 2]
            lo = k * _CH
            pending[k % 2] = (
                pltpu.async_copy(
                    t_hbm.at[idx_s.at[pl.ds(lo, _CH)]], buf_s, sem),
                pltpu.async_copy(
                    t_hbm.at[idx_d.at[pl.ds(lo, _CH)]], buf_r, sem),
            )

        def drain_pack_write(k):
            buf_s, buf_r, pk, _, sem_w = bufs[k % 2]
            for h in pending.pop(k % 2):
                h.wait()
            if k % 2 in wpending:
                wpending.pop(k % 2).wait()

            def body(j, carry):
                for hh in range(2):
                    a = buf_s[j, pl.ds(16 * hh, 16)]
                    b = buf_r[j, pl.ds(32 + 16 * hh, 16)]
                    pk[pl.ds(32 * j + 16 * hh, 16)] = a + b
                return carry

            lax.fori_loop(0, _CH, body, 0)
            fbase = (ebase + k * _CH) * latent
            wpending[k % 2] = pltpu.async_copy(
                pk, out_g.at[pl.ds(fbase, _CH * latent)], sem_w)

        fire(0)
        for k in range(1, nchunk):
            fire(k)
            drain_pack_write(k - 1)
        drain_pack_write(nchunk - 1)
        for h in wpending.values():
            h.wait()

    return gather_call


def kernel(node_attr, edge_index, edge_attr, global_attr, W1, b1, W2, b2):
    n_nodes, d_feat = node_attr.shape
    num_edges, d_edge = edge_attr.shape
    latent = W1.shape[1]
    d_out = W2.shape[1]

    src = edge_index[0].astype(jnp.int32)
    dst = edge_index[1].astype(jnp.int32)
    W1e = W1[:d_edge]
    W1s = W1[d_edge:d_edge + d_feat]
    W1r = W1[d_edge + d_feat:d_edge + 2 * d_feat]
    W1sr = jnp.concatenate([W1s, W1r], axis=1)
    W1g = W1[d_edge + 2 * d_feat:]

    T, c4 = pl.pallas_call(
        _prep_body,
        out_shape=[
            jax.ShapeDtypeStruct((n_nodes, 4 * latent), F32),
            jax.ShapeDtypeStruct((1, 4 * latent), F32),
        ],
    )(node_attr, W1sr, global_attr, W1g, b1.reshape(1, latent))

    G = _make_gather(num_edges, latent)(T, src, dst)

    eye4 = jnp.eye(4, dtype=F32)
    W1e_bd = jnp.kron(eye4, W1e)
    W2_bd = jnp.kron(eye4, W2)
    b24 = jnp.tile(b2, 4).reshape(1, 4 * d_out)

    rows = num_edges // 4
    block = 4000
    grid = rows // block
    out4 = pl.pallas_call(
        _mlp_body,
        grid=(grid,),
        in_specs=[
            pl.BlockSpec((block, 4 * d_edge), lambda i: (i, 0)),
            pl.BlockSpec((block * 4 * latent,), lambda i: (i,)),
            pl.BlockSpec((4 * d_edge, 4 * latent), lambda i: (0, 0)),
            pl.BlockSpec((1, 4 * latent), lambda i: (0, 0)),
            pl.BlockSpec((4 * latent, 4 * d_out), lambda i: (0, 0)),
            pl.BlockSpec((1, 4 * d_out), lambda i: (0, 0)),
        ],
        out_specs=pl.BlockSpec((block, 4 * d_out), lambda i: (i, 0)),
        out_shape=jax.ShapeDtypeStruct((rows, 4 * d_out), F32),
    )(edge_attr.reshape(rows, 4 * d_edge), G, W1e_bd, c4, W2_bd, b24)

    return out4.reshape(num_edges, d_out)
